# baseline (speedup 1.0000x reference)
import functools

import jax
import jax.numpy as jnp
from jax.experimental import pallas as pl
from jax.experimental.pallas import tpu as pltpu

EPS = 1e-05
RMS_EPS = 1e-06
TOP_K = 8
NUM_EXPERTS = 64


def _router_kernel(n_chunks, x_ref, nw_ref, gw_ref, probs_ref, idx_ref, logits_ref):
    CH = x_ref.shape[0] // n_chunks
    nw = nw_ref[...]
    gw = gw_ref[...]
    for c in range(n_chunks):
        lo = c * CH
        x = x_ref[lo:lo + CH, :]
        var = jnp.mean(x * x, axis=-1, keepdims=True)
        xn = x * jax.lax.rsqrt(var + RMS_EPS) * nw

        logits_t = jax.lax.dot_general(
            gw, xn, (((1,), (1,)), ((), ())),
            preferred_element_type=jnp.float32)
        logits_ref[lo:lo + CH, :] = logits_t.T

        m = jnp.max(logits_t, axis=0, keepdims=True)
        e = jnp.exp(logits_t - m)
        s_full = jnp.sum(e, axis=0, keepdims=True)

        iota = jax.lax.broadcasted_iota(jnp.int32, e.shape, 0)
        revkey = jnp.float32(NUM_EXPERTS - 1) - iota.astype(jnp.float32)

        work = e
        vals = []
        rkeys = []
        for k in range(TOP_K):
            mv = jnp.max(work, axis=0, keepdims=True)
            cand = jnp.where(work == mv, revkey, jnp.float32(-1.0))
            rk = jnp.max(cand, axis=0, keepdims=True)
            work = jnp.where(revkey == rk, -jnp.inf, work)
            vals.append(mv)
            rkeys.append(rk)

        topv_t = jnp.concatenate(vals, axis=0)
        rk_t = jnp.concatenate(rkeys, axis=0)
        denom = jnp.sum(topv_t, axis=0, keepdims=True) + jnp.float32(EPS) * s_full
        probs_ref[lo:lo + CH, :] = (topv_t / denom).T
        idx_ref[lo:lo + CH, :] = (
            (jnp.float32(NUM_EXPERTS - 1) - rk_t).astype(jnp.int32)).T


@functools.partial(jax.jit, static_argnames=())
def kernel(hidden_states, norm_weight, gate_weight):
    B, S, D = hidden_states.shape
    N = B * S
    E = gate_weight.shape[0]
    x = hidden_states.reshape(N, D)
    nw = norm_weight.reshape(1, D)

    TB = 2048
    N_CHUNKS = 4
    grid = (N // TB,)

    probs, idx, logits = pl.pallas_call(
        functools.partial(_router_kernel, N_CHUNKS),
        grid=grid,
        in_specs=[
            pl.BlockSpec((TB, D), lambda i: (i, 0)),
            pl.BlockSpec((1, D), lambda i: (0, 0)),
            pl.BlockSpec((E, D), lambda i: (0, 0)),
        ],
        out_specs=[
            pl.BlockSpec((TB, TOP_K), lambda i: (i, 0)),
            pl.BlockSpec((TB, TOP_K), lambda i: (i, 0)),
            pl.BlockSpec((TB, E), lambda i: (i, 0)),
        ],
        out_shape=[
            jax.ShapeDtypeStruct((N, TOP_K), jnp.float32),
            jax.ShapeDtypeStruct((N, TOP_K), jnp.int32),
            jax.ShapeDtypeStruct((N, E), jnp.float32),
        ],
        compiler_params=pltpu.CompilerParams(
            dimension_semantics=("parallel",),
        ),
    )(x, nw, gate_weight)
    return (probs, idx, logits)


# speedup vs baseline: 1.0400x; 1.0400x over previous
import functools

import jax
import jax.numpy as jnp
from jax.experimental import pallas as pl
from jax.experimental.pallas import tpu as pltpu

EPS = 1e-05
RMS_EPS = 1e-06
TOP_K = 8
NUM_EXPERTS = 64


def _router_kernel(n_chunks, x_ref, nw_ref, gw_ref, probs_ref, idx_ref, logits_ref):
    CH = x_ref.shape[0] // n_chunks
    nw = nw_ref[...]
    gw = gw_ref[...]
    for c in range(n_chunks):
        lo = c * CH
        x = x_ref[lo:lo + CH, :]
        var = jnp.mean(x * x, axis=-1, keepdims=True)
        xn = x * jax.lax.rsqrt(var + RMS_EPS) * nw

        logits_t = jax.lax.dot_general(
            gw, xn, (((1,), (1,)), ((), ())),
            preferred_element_type=jnp.float32)
        logits_ref[lo:lo + CH, :] = logits_t.T

        m = jnp.max(logits_t, axis=0, keepdims=True)
        e = jnp.exp(logits_t - m)
        s_full = jnp.sum(e, axis=0, keepdims=True)

        iota = jax.lax.broadcasted_iota(jnp.int32, e.shape, 0)
        revkey = jnp.float32(NUM_EXPERTS - 1) - iota.astype(jnp.float32)

        work = e
        vals = []
        rkeys = []
        for k in range(TOP_K):
            mv = jnp.max(work, axis=0, keepdims=True)
            cand = jnp.where(work == mv, revkey, jnp.float32(-1.0))
            rk = jnp.max(cand, axis=0, keepdims=True)
            work = jnp.where(revkey == rk, -jnp.inf, work)
            vals.append(mv)
            rkeys.append(rk)

        topv_t = jnp.concatenate(vals, axis=0)
        rk_t = jnp.concatenate(rkeys, axis=0)
        denom = jnp.sum(topv_t, axis=0, keepdims=True) + jnp.float32(EPS) * s_full
        probs_ref[lo:lo + CH, :] = (topv_t / denom).T
        idx_ref[lo:lo + CH, :] = (
            (jnp.float32(NUM_EXPERTS - 1) - rk_t).astype(jnp.int32)).T


@functools.partial(jax.jit, static_argnames=())
def kernel(hidden_states, norm_weight, gate_weight):
    B, S, D = hidden_states.shape
    N = B * S
    E = gate_weight.shape[0]
    x = hidden_states.reshape(N, D)
    nw = norm_weight.reshape(1, D)

    TB = 4096
    N_CHUNKS = 8
    grid = (N // TB,)

    probs, idx, logits = pl.pallas_call(
        functools.partial(_router_kernel, N_CHUNKS),
        grid=grid,
        in_specs=[
            pl.BlockSpec((TB, D), lambda i: (i, 0)),
            pl.BlockSpec((1, D), lambda i: (0, 0)),
            pl.BlockSpec((E, D), lambda i: (0, 0)),
        ],
        out_specs=[
            pl.BlockSpec((TB, TOP_K), lambda i: (i, 0)),
            pl.BlockSpec((TB, TOP_K), lambda i: (i, 0)),
            pl.BlockSpec((TB, E), lambda i: (i, 0)),
        ],
        out_shape=[
            jax.ShapeDtypeStruct((N, TOP_K), jnp.float32),
            jax.ShapeDtypeStruct((N, TOP_K), jnp.int32),
            jax.ShapeDtypeStruct((N, E), jnp.float32),
        ],
        compiler_params=pltpu.CompilerParams(
            dimension_semantics=("parallel",),
        ),
    )(x, nw, gate_weight)
    return (probs, idx, logits)
